# baseline (speedup 1.0000x reference)
import functools

import jax
import jax.numpy as jnp
from jax import lax
from jax.experimental import pallas as pl
from jax.experimental.pallas import tpu as pltpu
from jax.experimental.pallas import tpu_sc as plsc

NC = 2
NS = 16
EB = 128
ZR = 128

F32 = jnp.float32


def _pad_rows(a, np_rows):
    return jnp.pad(a, ((0, np_rows - a.shape[0]),) + ((0, 0),) * (a.ndim - 1))


def _deg_kernel(npad, epw):
    nchunks = epw // EB
    npw = npad // NS
    mesh = plsc.VectorSubcoreMesh(core_axis_name="c", subcore_axis_name="s")

    @functools.partial(
        pl.kernel,
        out_type=jax.ShapeDtypeStruct((NC * npad, 16), F32),
        mesh=mesh,
        scratch_types=[
            pltpu.VMEM((EB,), jnp.int32),
            pltpu.VMEM((EB,), jnp.int32),
            pltpu.VMEM((EB, 16), F32),
            pltpu.VMEM((ZR, 16), F32),
            pltpu.VMEM_SHARED((npad, 16), F32),
            pltpu.SemaphoreType.DMA,
            pltpu.SemaphoreType.DMA,
        ],
    )
    def deg(dst_hbm, out_hbm, didx0, didx1, ones_v, zbuf, acc, isem0, isem1):
        c = lax.axis_index("c")
        s = lax.axis_index("s")
        wid = s * NC + c
        didx = (didx0, didx1)
        isem = (isem0, isem1)
        zero16 = jnp.zeros((16,), F32)
        one16 = jnp.ones((16,), F32)

        def fill(r, _):
            zbuf[r, pl.ds(0, 16)] = zero16
            ones_v[r, pl.ds(0, 16)] = one16
            return 0

        lax.fori_loop(0, ZR, fill, 0)
        for t in range(npw // ZR):
            pltpu.sync_copy(zbuf, acc.at[pl.ds(s * npw + t * ZR, ZR)])
        plsc.subcore_barrier()

        base0 = wid * epw

        def issue_idx(j, slot):
            b = pl.multiple_of(base0 + j * EB, EB)
            pltpu.async_copy(dst_hbm.at[pl.ds(b, EB)], didx[slot], isem[slot])

        def wait_idx(slot):
            pltpu.make_async_copy(dst_hbm.at[pl.ds(0, EB)], didx[slot], isem[slot]).wait()

        issue_idx(0, 0)

        def pair(i, _):
            j = 2 * i
            issue_idx(j + 1, 1)
            wait_idx(0)
            pltpu.sync_copy(ones_v, acc.at[didx0], add=True)
            issue_idx(jnp.minimum(j + 2, nchunks - 1), 0)
            wait_idx(1)
            pltpu.sync_copy(ones_v, acc.at[didx1], add=True)
            return 0

        lax.fori_loop(0, nchunks // 2, pair, 0)
        wait_idx(0)
        plsc.subcore_barrier()
        for t in range(npw // ZR):
            r = s * npw + t * ZR
            pltpu.sync_copy(acc.at[pl.ds(r, ZR)], out_hbm.at[pl.ds(c * npad + r, ZR)])

    return deg


def _agg_kernel(npad, epw, K):
    nchunks = epw // EB
    npw = npad // NS
    mesh = plsc.VectorSubcoreMesh(core_axis_name="c", subcore_axis_name="s")

    @functools.partial(
        pl.kernel,
        out_type=jax.ShapeDtypeStruct((NC * npad, K), F32),
        mesh=mesh,
        scratch_types=[
            pltpu.VMEM((EB,), jnp.int32),
            pltpu.VMEM((EB,), jnp.int32),
            pltpu.VMEM((EB,), jnp.int32),
            pltpu.VMEM((EB,), jnp.int32),
            pltpu.VMEM((EB, K), F32),
            pltpu.VMEM((EB, K), F32),
            pltpu.VMEM_SHARED((npad, K), F32),
            pltpu.SemaphoreType.DMA,
            pltpu.SemaphoreType.DMA,
        ],
    )
    def agg(src_hbm, dst_hbm, h_hbm, out_hbm, sidx0, sidx1, didx0, didx1,
            rows0, rows1, acc, gsem0, gsem1):
        c = lax.axis_index("c")
        s = lax.axis_index("s")
        wid = s * NC + c
        sidx = (sidx0, sidx1)
        didx = (didx0, didx1)
        rows = (rows0, rows1)
        gsem = (gsem0, gsem1)
        zero16 = jnp.zeros((16,), F32)

        def fill(r, _):
            for j in range(K // 16):
                rows0[r, pl.ds(j * 16, 16)] = zero16
            return 0

        lax.fori_loop(0, ZR, fill, 0)
        for t in range(npw // ZR):
            pltpu.sync_copy(rows0, acc.at[pl.ds(s * npw + t * ZR, ZR)])
        plsc.subcore_barrier()

        base0 = wid * epw

        def load_idx_issue_gather(j, slot):
            b = pl.multiple_of(base0 + j * EB, EB)
            pltpu.sync_copy(src_hbm.at[pl.ds(b, EB)], sidx[slot])
            pltpu.sync_copy(dst_hbm.at[pl.ds(b, EB)], didx[slot])
            pltpu.async_copy(h_hbm.at[sidx[slot]], rows[slot], gsem[slot])

        def wait_gather(slot):
            pltpu.make_async_copy(h_hbm.at[sidx[slot]], rows[slot], gsem[slot]).wait()

        load_idx_issue_gather(0, 0)

        def pair(i, _):
            j = 2 * i
            load_idx_issue_gather(j + 1, 1)
            wait_gather(0)
            pltpu.sync_copy(rows0, acc.at[didx0], add=True)
            load_idx_issue_gather(jnp.minimum(j + 2, nchunks - 1), 0)
            wait_gather(1)
            pltpu.sync_copy(rows1, acc.at[didx1], add=True)
            return 0

        lax.fori_loop(0, nchunks // 2, pair, 0)
        wait_gather(0)
        plsc.subcore_barrier()
        for t in range(npw // ZR):
            r = s * npw + t * ZR
            pltpu.sync_copy(acc.at[pl.ds(r, ZR)], out_hbm.at[pl.ds(c * npad + r, ZR)])

    return agg


_DOT = functools.partial(jnp.dot, precision=lax.Precision.HIGHEST)


def _stage_a(dp, x):
    npad, df = x.shape

    def body(dp_ref, x_ref, dv_ref, hs_ref):
        deg = dp_ref[0:npad, 0:1] + dp_ref[npad : 2 * npad, 0:1] + 1.0
        dv = jnp.broadcast_to(lax.rsqrt(deg), (npad, df))
        dv_ref[...] = dv
        hs_ref[...] = x_ref[...] * dv

    return pl.pallas_call(
        body,
        out_shape=(
            jax.ShapeDtypeStruct((npad, df), F32),
            jax.ShapeDtypeStruct((npad, df), F32),
        ),
    )(dp, x)


def _stage_b(a1, x, dv, W1, b1, W2):
    npad, df = x.shape
    h2w = W2.shape[1]

    def body(a_ref, x_ref, dv_ref, w1_ref, b1_ref, w2_ref, m2_ref, hs_ref):
        dv = dv_ref[...]
        z1 = dv * (a_ref[0:npad, :] + a_ref[npad : 2 * npad, :]) + dv * dv * x_ref[...]
        h1 = jnp.maximum(_DOT(z1, w1_ref[...]) + b1_ref[...], 0.0)
        m2 = _DOT(h1, w2_ref[...])
        m2_ref[...] = m2
        hs_ref[...] = m2 * dv

    return pl.pallas_call(
        body,
        out_shape=(
            jax.ShapeDtypeStruct((npad, h2w), F32),
            jax.ShapeDtypeStruct((npad, h2w), F32),
        ),
    )(a1, x, dv, W1, b1, W2)


def _stage_c(a2, m2, dv, b2, g1, be1, W3, n_real):
    npad, h2w = m2.shape
    h3w = W3.shape[1]

    def body(a_ref, m2_ref, dv_ref, b2_ref, g1_ref, be1_ref, w3_ref, m3_ref, hs_ref):
        dv = dv_ref[...]
        m2v = m2_ref[...]
        z2 = dv * (a_ref[0:npad, :] + a_ref[npad : 2 * npad, :]) + dv * dv * m2v + b2_ref[...]
        h2 = jnp.maximum(z2, 0.0)
        mask = (lax.broadcasted_iota(jnp.int32, (npad, h2w), 0) < n_real).astype(F32)
        mean = jnp.sum(h2 * mask, axis=0, keepdims=True) / n_real
        d = (h2 - mean) * mask
        var = jnp.sum(d * d, axis=0, keepdims=True) / n_real
        h2n = (h2 - mean) * lax.rsqrt(var + 1e-5) * g1_ref[...] + be1_ref[...]
        m3 = _DOT(h2n, w3_ref[...])
        m3_ref[...] = m3
        hs_ref[...] = jnp.concatenate(
            [m3 * dv_ref[0:npad, 0:h3w], jnp.zeros((npad, h2w - h3w), F32)], axis=1
        )

    return pl.pallas_call(
        body,
        out_shape=(
            jax.ShapeDtypeStruct((npad, h3w), F32),
            jax.ShapeDtypeStruct((npad, h2w), F32),
        ),
    )(a2, m2, dv, b2, g1, be1, W3)


def _stage_d(a3, m3, dv, b3, g2, be2, bT, fcW, fcb, n_real):
    npad, h3w = m3.shape
    g = bT.shape[0]
    c_out = fcW.shape[1]

    def body(a_ref, m3_ref, dv_ref, b3_ref, g2_ref, be2_ref, bT_ref, fcW_ref, fcb_ref, o_ref):
        dv = dv_ref[0:npad, 0:h3w]
        m3v = m3_ref[...]
        z3 = (dv * (a_ref[0:npad, 0:h3w] + a_ref[npad : 2 * npad, 0:h3w])
              + dv * dv * m3v + b3_ref[...])
        h3 = jnp.maximum(z3, 0.0)
        mask = (lax.broadcasted_iota(jnp.int32, (npad, h3w), 0) < n_real).astype(F32)
        mean = jnp.sum(h3 * mask, axis=0, keepdims=True) / n_real
        d = (h3 - mean) * mask
        var = jnp.sum(d * d, axis=0, keepdims=True) / n_real
        h3n = (h3 - mean) * lax.rsqrt(var + 1e-5) * g2_ref[...] + be2_ref[...]
        gid = lax.broadcasted_iota(jnp.int32, (g, npad), 0)
        MT = (bT_ref[...] == gid).astype(F32)
        seg = _DOT(MT, h3n)
        cnt = jnp.sum(MT, axis=1, keepdims=True)
        p = seg / jnp.maximum(cnt, 1.0)
        o_ref[...] = _DOT(p, fcW_ref[...]) + fcb_ref[...]

    return pl.pallas_call(
        body,
        out_shape=jax.ShapeDtypeStruct((g, c_out), F32),
    )(a3, m3, dv, b3, g2, be2, bT, fcW, fcb)


def kernel(x, edge_index, batch, W1, b1, W2, b2, g1, be1, W3, b3, g2, be2, fcW, fcb):
    n, df = x.shape
    e = edge_index.shape[1]
    g = 64
    h2w = W2.shape[1]
    h3w = W3.shape[1]

    npad = -(-n // (NS * ZR)) * (NS * ZR)
    nw = NC * NS
    epad = -(-e // (nw * EB * 2)) * (nw * EB * 2)
    epw = epad // nw

    src = jnp.pad(edge_index[0], (0, epad - e), constant_values=n)
    dst = jnp.pad(edge_index[1], (0, epad - e), constant_values=n)
    x_p = _pad_rows(x, npad)
    bT = jnp.broadcast_to(
        jnp.pad(batch, (0, npad - n), constant_values=-1)[None, :], (g, npad)
    )

    aggs = {}

    def _agg(h):
        k = h.shape[1]
        if k not in aggs:
            aggs[k] = _agg_kernel(npad, epw, k)
        return aggs[k](src, dst, h)

    deg_p = _deg_kernel(npad, epw)(dst)
    dv, hs1 = _stage_a(deg_p, x_p)

    a1 = _agg(hs1)
    m2, hs2 = _stage_b(a1, x_p, dv, W1, b1.reshape(1, -1), W2)

    a2 = _agg(hs2)
    m3, hs3 = _stage_c(a2, m2, dv, b2.reshape(1, -1), g1.reshape(1, -1),
                       be1.reshape(1, -1), W3, n)

    a3 = _agg(hs3)
    out = _stage_d(a3, m3, dv, b3.reshape(1, -1), g2.reshape(1, -1),
                   be2.reshape(1, -1), bT, fcW, fcb.reshape(1, -1), n)
    return out


# speedup vs baseline: 1.2550x; 1.2550x over previous
import functools

import jax
import jax.numpy as jnp
from jax import lax
from jax.experimental import pallas as pl
from jax.experimental.pallas import tpu as pltpu
from jax.experimental.pallas import tpu_sc as plsc

NC = 2
NS = 16
EB = 128
ZR = 128
FRAC_C0 = 0.28

F32 = jnp.float32


def _pad_rows(a, np_rows):
    return jnp.pad(a, ((0, np_rows - a.shape[0]),) + ((0, 0),) * (a.ndim - 1))


def _deg_kernel(npad, nch0, nch1):
    npw = npad // NS
    mesh = plsc.VectorSubcoreMesh(core_axis_name="c", subcore_axis_name="s")

    @functools.partial(
        pl.kernel,
        out_type=jax.ShapeDtypeStruct((NC * npad, 16), F32),
        mesh=mesh,
        scratch_types=[
            pltpu.VMEM((EB,), jnp.int32),
            pltpu.VMEM((EB,), jnp.int32),
            pltpu.VMEM((EB, 16), F32),
            pltpu.VMEM((ZR, 16), F32),
            pltpu.VMEM_SHARED((npad, 16), F32),
            pltpu.SemaphoreType.DMA,
            pltpu.SemaphoreType.DMA,
        ],
    )
    def deg(dst_hbm, out_hbm, didx0, didx1, ones_v, zbuf, acc, isem0, isem1):
        c = lax.axis_index("c")
        s = lax.axis_index("s")
        didx = (didx0, didx1)
        isem = (isem0, isem1)
        zero16 = jnp.zeros((16,), F32)
        one16 = jnp.ones((16,), F32)

        def fill(r, _):
            zbuf[r, pl.ds(0, 16)] = zero16
            ones_v[r, pl.ds(0, 16)] = one16
            return 0

        lax.fori_loop(0, ZR, fill, 0)
        for t in range(npw // ZR):
            pltpu.sync_copy(zbuf, acc.at[pl.ds(s * npw + t * ZR, ZR)])
        plsc.subcore_barrier()

        m = jnp.where(c == 0, nch0 // NS, nch1 // NS)
        base0 = jnp.where(c == 0, 0, nch0) + s * m
        last = base0 + m - 1

        def issue_idx(j, slot):
            b = pl.multiple_of(j * EB, EB)
            pltpu.async_copy(dst_hbm.at[pl.ds(b, EB)], didx[slot], isem[slot])

        def wait_idx(slot):
            pltpu.make_async_copy(dst_hbm.at[pl.ds(0, EB)], didx[slot], isem[slot]).wait()

        issue_idx(base0, 0)

        def pair(i, _):
            j = base0 + 2 * i
            issue_idx(j + 1, 1)
            wait_idx(0)
            pltpu.sync_copy(ones_v, acc.at[didx0], add=True)
            issue_idx(jnp.minimum(j + 2, last), 0)
            wait_idx(1)
            pltpu.sync_copy(ones_v, acc.at[didx1], add=True)
            return 0

        lax.fori_loop(0, m // 2, pair, 0)
        wait_idx(0)
        plsc.subcore_barrier()
        for t in range(npw // ZR):
            r = s * npw + t * ZR
            pltpu.sync_copy(acc.at[pl.ds(r, ZR)], out_hbm.at[pl.ds(c * npad + r, ZR)])

    return deg


def _agg_kernel(npad, nch0, nch1, K):
    npw = npad // NS
    mesh = plsc.VectorSubcoreMesh(core_axis_name="c", subcore_axis_name="s")

    @functools.partial(
        pl.kernel,
        out_type=jax.ShapeDtypeStruct((NC * npad, K), F32),
        mesh=mesh,
        scratch_types=[
            pltpu.VMEM((EB,), jnp.int32),
            pltpu.VMEM((EB,), jnp.int32),
            pltpu.VMEM((EB,), jnp.int32),
            pltpu.VMEM((EB,), jnp.int32),
            pltpu.VMEM((EB, K), F32),
            pltpu.VMEM((EB, K), F32),
            pltpu.VMEM_SHARED((npad, K), F32),
            pltpu.SemaphoreType.DMA,
            pltpu.SemaphoreType.DMA,
        ],
    )
    def agg(src_hbm, dst_hbm, h_hbm, out_hbm, sidx0, sidx1, didx0, didx1,
            rows0, rows1, acc, gsem0, gsem1):
        c = lax.axis_index("c")
        s = lax.axis_index("s")
        sidx = (sidx0, sidx1)
        didx = (didx0, didx1)
        rows = (rows0, rows1)
        gsem = (gsem0, gsem1)
        zero16 = jnp.zeros((16,), F32)

        def fill(r, _):
            for j in range(K // 16):
                rows0[r, pl.ds(j * 16, 16)] = zero16
            return 0

        lax.fori_loop(0, ZR, fill, 0)
        for t in range(npw // ZR):
            pltpu.sync_copy(rows0, acc.at[pl.ds(s * npw + t * ZR, ZR)])
        plsc.subcore_barrier()

        m = jnp.where(c == 0, nch0 // NS, nch1 // NS)
        base0 = jnp.where(c == 0, 0, nch0) + s * m
        last = base0 + m - 1

        def load_idx_issue_gather(j, slot):
            b = pl.multiple_of(j * EB, EB)
            pltpu.sync_copy(src_hbm.at[pl.ds(b, EB)], sidx[slot])
            pltpu.sync_copy(dst_hbm.at[pl.ds(b, EB)], didx[slot])
            pltpu.async_copy(h_hbm.at[sidx[slot]], rows[slot], gsem[slot])

        def wait_gather(slot):
            pltpu.make_async_copy(h_hbm.at[sidx[slot]], rows[slot], gsem[slot]).wait()

        load_idx_issue_gather(base0, 0)

        def pair(i, _):
            j = base0 + 2 * i
            load_idx_issue_gather(j + 1, 1)
            wait_gather(0)
            pltpu.sync_copy(rows0, acc.at[didx0], add=True)
            load_idx_issue_gather(jnp.minimum(j + 2, last), 0)
            wait_gather(1)
            pltpu.sync_copy(rows1, acc.at[didx1], add=True)
            return 0

        lax.fori_loop(0, m // 2, pair, 0)
        wait_gather(0)
        plsc.subcore_barrier()
        for t in range(npw // ZR):
            r = s * npw + t * ZR
            pltpu.sync_copy(acc.at[pl.ds(r, ZR)], out_hbm.at[pl.ds(c * npad + r, ZR)])

    return agg


_DOT = functools.partial(jnp.dot, precision=lax.Precision.HIGHEST)


def _stage_a(dp, x):
    npad, df = x.shape

    def body(dp_ref, x_ref, dv_ref, hs_ref):
        deg = dp_ref[0:npad, 0:1] + dp_ref[npad : 2 * npad, 0:1] + 1.0
        dv = jnp.broadcast_to(lax.rsqrt(deg), (npad, df))
        dv_ref[...] = dv
        hs_ref[...] = x_ref[...] * dv

    return pl.pallas_call(
        body,
        out_shape=(
            jax.ShapeDtypeStruct((npad, df), F32),
            jax.ShapeDtypeStruct((npad, df), F32),
        ),
    )(dp, x)


def _stage_b(a1, x, dv, W1, b1, W2):
    npad, df = x.shape
    h2w = W2.shape[1]

    def body(a_ref, x_ref, dv_ref, w1_ref, b1_ref, w2_ref, m2_ref, hs_ref):
        dv = dv_ref[...]
        z1 = dv * (a_ref[0:npad, :] + a_ref[npad : 2 * npad, :]) + dv * dv * x_ref[...]
        h1 = jnp.maximum(_DOT(z1, w1_ref[...]) + b1_ref[...], 0.0)
        m2 = _DOT(h1, w2_ref[...])
        m2_ref[...] = m2
        hs_ref[...] = m2 * dv

    return pl.pallas_call(
        body,
        out_shape=(
            jax.ShapeDtypeStruct((npad, h2w), F32),
            jax.ShapeDtypeStruct((npad, h2w), F32),
        ),
    )(a1, x, dv, W1, b1, W2)


def _stage_c(a2, m2, dv, b2, g1, be1, W3, n_real):
    npad, h2w = m2.shape
    h3w = W3.shape[1]

    def body(a_ref, m2_ref, dv_ref, b2_ref, g1_ref, be1_ref, w3_ref, m3_ref, hs_ref):
        dv = dv_ref[...]
        m2v = m2_ref[...]
        z2 = dv * (a_ref[0:npad, :] + a_ref[npad : 2 * npad, :]) + dv * dv * m2v + b2_ref[...]
        h2 = jnp.maximum(z2, 0.0)
        mask = (lax.broadcasted_iota(jnp.int32, (npad, h2w), 0) < n_real).astype(F32)
        mean = jnp.sum(h2 * mask, axis=0, keepdims=True) / n_real
        d = (h2 - mean) * mask
        var = jnp.sum(d * d, axis=0, keepdims=True) / n_real
        h2n = (h2 - mean) * lax.rsqrt(var + 1e-5) * g1_ref[...] + be1_ref[...]
        m3 = _DOT(h2n, w3_ref[...])
        m3_ref[...] = m3
        hs_ref[...] = jnp.concatenate(
            [m3 * dv_ref[0:npad, 0:h3w], jnp.zeros((npad, h2w - h3w), F32)], axis=1
        )

    return pl.pallas_call(
        body,
        out_shape=(
            jax.ShapeDtypeStruct((npad, h3w), F32),
            jax.ShapeDtypeStruct((npad, h2w), F32),
        ),
    )(a2, m2, dv, b2, g1, be1, W3)


def _stage_d(a3, m3, dv, b3, g2, be2, bT, fcW, fcb, n_real):
    npad, h3w = m3.shape
    g = bT.shape[0]
    c_out = fcW.shape[1]

    def body(a_ref, m3_ref, dv_ref, b3_ref, g2_ref, be2_ref, bT_ref, fcW_ref, fcb_ref, o_ref):
        dv = dv_ref[0:npad, 0:h3w]
        m3v = m3_ref[...]
        z3 = (dv * (a_ref[0:npad, 0:h3w] + a_ref[npad : 2 * npad, 0:h3w])
              + dv * dv * m3v + b3_ref[...])
        h3 = jnp.maximum(z3, 0.0)
        mask = (lax.broadcasted_iota(jnp.int32, (npad, h3w), 0) < n_real).astype(F32)
        mean = jnp.sum(h3 * mask, axis=0, keepdims=True) / n_real
        d = (h3 - mean) * mask
        var = jnp.sum(d * d, axis=0, keepdims=True) / n_real
        h3n = (h3 - mean) * lax.rsqrt(var + 1e-5) * g2_ref[...] + be2_ref[...]
        gid = lax.broadcasted_iota(jnp.int32, (g, npad), 0)
        MT = (bT_ref[...] == gid).astype(F32)
        seg = _DOT(MT, h3n)
        cnt = jnp.sum(MT, axis=1, keepdims=True)
        p = seg / jnp.maximum(cnt, 1.0)
        o_ref[...] = _DOT(p, fcW_ref[...]) + fcb_ref[...]

    return pl.pallas_call(
        body,
        out_shape=jax.ShapeDtypeStruct((g, c_out), F32),
    )(a3, m3, dv, b3, g2, be2, bT, fcW, fcb)


def kernel(x, edge_index, batch, W1, b1, W2, b2, g1, be1, W3, b3, g2, be2, fcW, fcb):
    n, df = x.shape
    e = edge_index.shape[1]
    g = 64
    h2w = W2.shape[1]
    h3w = W3.shape[1]

    npad = -(-n // (NS * ZR)) * (NS * ZR)
    grp = 2 * NS * EB
    epad = -(-e // grp) * grp
    nch = epad // EB
    nch0 = min(max(round(FRAC_C0 * nch / (2 * NS)) * 2 * NS, 2 * NS), nch - 2 * NS)
    nch1 = nch - nch0

    src = jnp.pad(edge_index[0], (0, epad - e), constant_values=n)
    dst = jnp.pad(edge_index[1], (0, epad - e), constant_values=n)
    x_p = _pad_rows(x, npad)
    bT = jnp.broadcast_to(
        jnp.pad(batch, (0, npad - n), constant_values=-1)[None, :], (g, npad)
    )

    aggs = {}

    def _agg(h):
        k = h.shape[1]
        if k not in aggs:
            aggs[k] = _agg_kernel(npad, nch0, nch1, k)
        return aggs[k](src, dst, h)

    deg_p = _deg_kernel(npad, nch0, nch1)(dst)
    dv, hs1 = _stage_a(deg_p, x_p)

    a1 = _agg(hs1)
    m2, hs2 = _stage_b(a1, x_p, dv, W1, b1.reshape(1, -1), W2)

    a2 = _agg(hs2)
    m3, hs3 = _stage_c(a2, m2, dv, b2.reshape(1, -1), g1.reshape(1, -1),
                       be1.reshape(1, -1), W3, n)

    a3 = _agg(hs3)
    out = _stage_d(a3, m3, dv, b3.reshape(1, -1), g2.reshape(1, -1),
                   be2.reshape(1, -1), bT, fcW, fcb.reshape(1, -1), n)
    return out


# speedup vs baseline: 1.2559x; 1.0007x over previous
import functools

import jax
import jax.numpy as jnp
from jax import lax
from jax.experimental import pallas as pl
from jax.experimental.pallas import tpu as pltpu
from jax.experimental.pallas import tpu_sc as plsc

NC = 2
NS = 16
EB = 128
ZR = 128
FRAC_C0 = 0.28

F32 = jnp.float32


def _pad_rows(a, np_rows):
    return jnp.pad(a, ((0, np_rows - a.shape[0]),) + ((0, 0),) * (a.ndim - 1))


def _deg_kernel(npad, m0, m1):
    npw = npad // NS
    mesh = plsc.VectorSubcoreMesh(core_axis_name="c", subcore_axis_name="s")

    @functools.partial(
        pl.kernel,
        out_type=jax.ShapeDtypeStruct((NC * npad, 16), F32),
        mesh=mesh,
        scratch_types=[
            pltpu.VMEM((EB,), jnp.int32),
            pltpu.VMEM((EB,), jnp.int32),
            pltpu.VMEM((EB, 16), F32),
            pltpu.VMEM((ZR, 16), F32),
            pltpu.VMEM_SHARED((npad, 16), F32),
            pltpu.SemaphoreType.DMA,
            pltpu.SemaphoreType.DMA,
        ],
    )
    def deg(dst_hbm, out_hbm, didx0, didx1, ones_v, zbuf, acc, isem0, isem1):
        c = lax.axis_index("c")
        s = lax.axis_index("s")
        didx = (didx0, didx1)
        isem = (isem0, isem1)
        zero16 = jnp.zeros((16,), F32)
        one16 = jnp.ones((16,), F32)

        def fill(r, _):
            zbuf[r, pl.ds(0, 16)] = zero16
            ones_v[r, pl.ds(0, 16)] = one16
            return 0

        lax.fori_loop(0, ZR, fill, 0)
        for t in range(npw // ZR):
            pltpu.sync_copy(zbuf, acc.at[pl.ds(s * npw + t * ZR, ZR)])
        plsc.subcore_barrier()

        base0 = s * ((m0 + m1) * EB) + c * (m0 * EB)

        def issue_idx(j, slot):
            b = pl.multiple_of(base0 + j * EB, EB)
            pltpu.async_copy(dst_hbm.at[pl.ds(b, EB)], didx[slot], isem[slot])

        def wait_idx(slot):
            pltpu.make_async_copy(dst_hbm.at[pl.ds(0, EB)], didx[slot], isem[slot]).wait()

        def runloop(mch):
            issue_idx(0, 0)

            def pair(i, _):
                j = 2 * i
                issue_idx(j + 1, 1)
                wait_idx(0)
                pltpu.sync_copy(ones_v, acc.at[didx0], add=True)
                issue_idx(jnp.minimum(j + 2, mch - 1), 0)
                wait_idx(1)
                pltpu.sync_copy(ones_v, acc.at[didx1], add=True)
                return 0

            lax.fori_loop(0, mch // 2, pair, 0)
            wait_idx(0)

        @pl.when(c == 0)
        def _():
            runloop(m0)

        @pl.when(c == 1)
        def _():
            runloop(m1)
        plsc.subcore_barrier()
        for t in range(npw // ZR):
            r = s * npw + t * ZR
            pltpu.sync_copy(acc.at[pl.ds(r, ZR)], out_hbm.at[pl.ds(c * npad + r, ZR)])

    return deg


def _agg_kernel(npad, m0, m1, K):
    npw = npad // NS
    mesh = plsc.VectorSubcoreMesh(core_axis_name="c", subcore_axis_name="s")

    @functools.partial(
        pl.kernel,
        out_type=jax.ShapeDtypeStruct((NC * npad, K), F32),
        mesh=mesh,
        scratch_types=[
            pltpu.VMEM((EB,), jnp.int32),
            pltpu.VMEM((EB,), jnp.int32),
            pltpu.VMEM((EB,), jnp.int32),
            pltpu.VMEM((EB,), jnp.int32),
            pltpu.VMEM((EB, K), F32),
            pltpu.VMEM((EB, K), F32),
            pltpu.VMEM_SHARED((npad, K), F32),
            pltpu.SemaphoreType.DMA,
            pltpu.SemaphoreType.DMA,
        ],
    )
    def agg(src_hbm, dst_hbm, h_hbm, out_hbm, sidx0, sidx1, didx0, didx1,
            rows0, rows1, acc, gsem0, gsem1):
        c = lax.axis_index("c")
        s = lax.axis_index("s")
        sidx = (sidx0, sidx1)
        didx = (didx0, didx1)
        rows = (rows0, rows1)
        gsem = (gsem0, gsem1)
        zero16 = jnp.zeros((16,), F32)

        def fill(r, _):
            for j in range(K // 16):
                rows0[r, pl.ds(j * 16, 16)] = zero16
            return 0

        lax.fori_loop(0, ZR, fill, 0)
        for t in range(npw // ZR):
            pltpu.sync_copy(rows0, acc.at[pl.ds(s * npw + t * ZR, ZR)])
        plsc.subcore_barrier()

        base0 = s * ((m0 + m1) * EB) + c * (m0 * EB)

        def load_idx_issue_gather(j, slot):
            b = pl.multiple_of(base0 + j * EB, EB)
            pltpu.sync_copy(src_hbm.at[pl.ds(b, EB)], sidx[slot])
            pltpu.sync_copy(dst_hbm.at[pl.ds(b, EB)], didx[slot])
            pltpu.async_copy(h_hbm.at[sidx[slot]], rows[slot], gsem[slot])

        def wait_gather(slot):
            pltpu.make_async_copy(h_hbm.at[sidx[slot]], rows[slot], gsem[slot]).wait()

        def runloop(mch):
            load_idx_issue_gather(0, 0)

            def pair(i, _):
                j = 2 * i
                load_idx_issue_gather(j + 1, 1)
                wait_gather(0)
                pltpu.sync_copy(rows0, acc.at[didx0], add=True)
                load_idx_issue_gather(jnp.minimum(j + 2, mch - 1), 0)
                wait_gather(1)
                pltpu.sync_copy(rows1, acc.at[didx1], add=True)
                return 0

            lax.fori_loop(0, mch // 2, pair, 0)
            wait_gather(0)

        @pl.when(c == 0)
        def _():
            runloop(m0)

        @pl.when(c == 1)
        def _():
            runloop(m1)
        plsc.subcore_barrier()
        for t in range(npw // ZR):
            r = s * npw + t * ZR
            pltpu.sync_copy(acc.at[pl.ds(r, ZR)], out_hbm.at[pl.ds(c * npad + r, ZR)])

    return agg


_DOT = functools.partial(jnp.dot, precision=lax.Precision.HIGHEST)


def _stage_a(dp, x):
    npad, df = x.shape

    def body(dp_ref, x_ref, dv_ref, hs_ref):
        deg = dp_ref[0:npad, 0:1] + dp_ref[npad : 2 * npad, 0:1] + 1.0
        dv = jnp.broadcast_to(lax.rsqrt(deg), (npad, df))
        dv_ref[...] = dv
        hs_ref[...] = x_ref[...] * dv

    return pl.pallas_call(
        body,
        out_shape=(
            jax.ShapeDtypeStruct((npad, df), F32),
            jax.ShapeDtypeStruct((npad, df), F32),
        ),
    )(dp, x)


def _stage_b(a1, x, dv, W1, b1, W2):
    npad, df = x.shape
    h2w = W2.shape[1]

    def body(a_ref, x_ref, dv_ref, w1_ref, b1_ref, w2_ref, m2_ref, hs_ref):
        dv = dv_ref[...]
        z1 = dv * (a_ref[0:npad, :] + a_ref[npad : 2 * npad, :]) + dv * dv * x_ref[...]
        h1 = jnp.maximum(_DOT(z1, w1_ref[...]) + b1_ref[...], 0.0)
        m2 = _DOT(h1, w2_ref[...])
        m2_ref[...] = m2
        hs_ref[...] = m2 * dv

    return pl.pallas_call(
        body,
        out_shape=(
            jax.ShapeDtypeStruct((npad, h2w), F32),
            jax.ShapeDtypeStruct((npad, h2w), F32),
        ),
    )(a1, x, dv, W1, b1, W2)


def _stage_c(a2, m2, dv, b2, g1, be1, W3, n_real):
    npad, h2w = m2.shape
    h3w = W3.shape[1]

    def body(a_ref, m2_ref, dv_ref, b2_ref, g1_ref, be1_ref, w3_ref, m3_ref, hs_ref):
        dv = dv_ref[...]
        m2v = m2_ref[...]
        z2 = dv * (a_ref[0:npad, :] + a_ref[npad : 2 * npad, :]) + dv * dv * m2v + b2_ref[...]
        h2 = jnp.maximum(z2, 0.0)
        mask = (lax.broadcasted_iota(jnp.int32, (npad, h2w), 0) < n_real).astype(F32)
        mean = jnp.sum(h2 * mask, axis=0, keepdims=True) / n_real
        d = (h2 - mean) * mask
        var = jnp.sum(d * d, axis=0, keepdims=True) / n_real
        h2n = (h2 - mean) * lax.rsqrt(var + 1e-5) * g1_ref[...] + be1_ref[...]
        m3 = _DOT(h2n, w3_ref[...])
        m3_ref[...] = m3
        hs_ref[...] = jnp.concatenate(
            [m3 * dv_ref[0:npad, 0:h3w], jnp.zeros((npad, h2w - h3w), F32)], axis=1
        )

    return pl.pallas_call(
        body,
        out_shape=(
            jax.ShapeDtypeStruct((npad, h3w), F32),
            jax.ShapeDtypeStruct((npad, h2w), F32),
        ),
    )(a2, m2, dv, b2, g1, be1, W3)


def _stage_d(a3, m3, dv, b3, g2, be2, bT, fcW, fcb, n_real):
    npad, h3w = m3.shape
    g = bT.shape[0]
    c_out = fcW.shape[1]

    def body(a_ref, m3_ref, dv_ref, b3_ref, g2_ref, be2_ref, bT_ref, fcW_ref, fcb_ref, o_ref):
        dv = dv_ref[0:npad, 0:h3w]
        m3v = m3_ref[...]
        z3 = (dv * (a_ref[0:npad, 0:h3w] + a_ref[npad : 2 * npad, 0:h3w])
              + dv * dv * m3v + b3_ref[...])
        h3 = jnp.maximum(z3, 0.0)
        mask = (lax.broadcasted_iota(jnp.int32, (npad, h3w), 0) < n_real).astype(F32)
        mean = jnp.sum(h3 * mask, axis=0, keepdims=True) / n_real
        d = (h3 - mean) * mask
        var = jnp.sum(d * d, axis=0, keepdims=True) / n_real
        h3n = (h3 - mean) * lax.rsqrt(var + 1e-5) * g2_ref[...] + be2_ref[...]
        gid = lax.broadcasted_iota(jnp.int32, (g, npad), 0)
        MT = (bT_ref[...] == gid).astype(F32)
        seg = _DOT(MT, h3n)
        cnt = jnp.sum(MT, axis=1, keepdims=True)
        p = seg / jnp.maximum(cnt, 1.0)
        o_ref[...] = _DOT(p, fcW_ref[...]) + fcb_ref[...]

    return pl.pallas_call(
        body,
        out_shape=jax.ShapeDtypeStruct((g, c_out), F32),
    )(a3, m3, dv, b3, g2, be2, bT, fcW, fcb)


def kernel(x, edge_index, batch, W1, b1, W2, b2, g1, be1, W3, b3, g2, be2, fcW, fcb):
    n, df = x.shape
    e = edge_index.shape[1]
    g = 64
    h2w = W2.shape[1]
    h3w = W3.shape[1]

    npad = -(-n // (NS * ZR)) * (NS * ZR)
    cps = 2 * (-(-e // (EB * 2 * NS)))
    m0 = min(max(round(FRAC_C0 * cps / 2) * 2, 2), cps - 2)
    m1 = cps - m0
    epad = NS * cps * EB

    src = jnp.pad(edge_index[0], (0, epad - e), constant_values=n)
    dst = jnp.pad(edge_index[1], (0, epad - e), constant_values=n)
    x_p = _pad_rows(x, npad)
    bT = jnp.broadcast_to(
        jnp.pad(batch, (0, npad - n), constant_values=-1)[None, :], (g, npad)
    )

    aggs = {}

    def _agg(h):
        k = h.shape[1]
        if k not in aggs:
            aggs[k] = _agg_kernel(npad, m0, m1, k)
        return aggs[k](src, dst, h)

    deg_p = _deg_kernel(npad, m0, m1)(dst)
    dv, hs1 = _stage_a(deg_p, x_p)

    a1 = _agg(hs1)
    m2, hs2 = _stage_b(a1, x_p, dv, W1, b1.reshape(1, -1), W2)

    a2 = _agg(hs2)
    m3, hs3 = _stage_c(a2, m2, dv, b2.reshape(1, -1), g1.reshape(1, -1),
                       be1.reshape(1, -1), W3, n)

    a3 = _agg(hs3)
    out = _stage_d(a3, m3, dv, b3.reshape(1, -1), g2.reshape(1, -1),
                   be2.reshape(1, -1), bT, fcW, fcb.reshape(1, -1), n)
    return out


# speedup vs baseline: 1.4965x; 1.1916x over previous
import functools

import jax
import jax.numpy as jnp
from jax import lax
from jax.experimental import pallas as pl
from jax.experimental.pallas import tpu as pltpu
from jax.experimental.pallas import tpu_sc as plsc

NC = 2
NS = 16
EB = 128
ZR = 128
FRAC_C0 = 0.58

F32 = jnp.float32


def _pad_rows(a, np_rows):
    return jnp.pad(a, ((0, np_rows - a.shape[0]),) + ((0, 0),) * (a.ndim - 1))


def _deg_kernel(npad, m0, m1):
    npw = npad // NS
    mesh = plsc.VectorSubcoreMesh(core_axis_name="c", subcore_axis_name="s")

    @functools.partial(
        pl.kernel,
        out_type=jax.ShapeDtypeStruct((NC * npad, 16), F32),
        mesh=mesh,
        scratch_types=[
            pltpu.VMEM((EB,), jnp.int32),
            pltpu.VMEM((EB,), jnp.int32),
            pltpu.VMEM((EB, 16), F32),
            pltpu.VMEM((ZR, 16), F32),
            pltpu.VMEM_SHARED((npad, 16), F32),
            pltpu.SemaphoreType.DMA,
            pltpu.SemaphoreType.DMA,
        ],
    )
    def deg(dst_hbm, out_hbm, didx0, didx1, ones_v, zbuf, acc, isem0, isem1):
        c = lax.axis_index("c")
        s = lax.axis_index("s")
        didx = (didx0, didx1)
        isem = (isem0, isem1)
        zero16 = jnp.zeros((16,), F32)
        one16 = jnp.ones((16,), F32)

        def fill(r, _):
            zbuf[r, pl.ds(0, 16)] = zero16
            ones_v[r, pl.ds(0, 16)] = one16
            return 0

        lax.fori_loop(0, ZR, fill, 0)
        for t in range(npw // ZR):
            pltpu.sync_copy(zbuf, acc.at[pl.ds(s * npw + t * ZR, ZR)])
        plsc.subcore_barrier()

        base0 = s * ((m0 + m1) * EB) + c * (m0 * EB)

        def issue_idx(j, slot):
            b = pl.multiple_of(base0 + j * EB, EB)
            pltpu.async_copy(dst_hbm.at[pl.ds(b, EB)], didx[slot], isem[slot])

        def wait_idx(slot):
            pltpu.make_async_copy(dst_hbm.at[pl.ds(0, EB)], didx[slot], isem[slot]).wait()

        def runloop(mch):
            issue_idx(0, 0)

            def pair(i, _):
                j = 2 * i
                issue_idx(j + 1, 1)
                wait_idx(0)
                pltpu.sync_copy(ones_v, acc.at[didx0], add=True)
                issue_idx(jnp.minimum(j + 2, mch - 1), 0)
                wait_idx(1)
                pltpu.sync_copy(ones_v, acc.at[didx1], add=True)
                return 0

            lax.fori_loop(0, mch // 2, pair, 0)
            wait_idx(0)

        @pl.when(c == 0)
        def _():
            runloop(m0)

        @pl.when(c == 1)
        def _():
            runloop(m1)
        plsc.subcore_barrier()
        for t in range(npw // ZR):
            r = s * npw + t * ZR
            pltpu.sync_copy(acc.at[pl.ds(r, ZR)], out_hbm.at[pl.ds(c * npad + r, ZR)])

    return deg


def _agg_kernel(npad, m0, m1, K):
    npw = npad // NS
    mesh = plsc.VectorSubcoreMesh(core_axis_name="c", subcore_axis_name="s")

    @functools.partial(
        pl.kernel,
        out_type=jax.ShapeDtypeStruct((NC * npad, K), F32),
        mesh=mesh,
        scratch_types=[
            pltpu.VMEM((EB,), jnp.int32),
            pltpu.VMEM((EB,), jnp.int32),
            pltpu.VMEM((EB,), jnp.int32),
            pltpu.VMEM((EB,), jnp.int32),
            pltpu.VMEM((EB, K), F32),
            pltpu.VMEM((EB, K), F32),
            pltpu.VMEM_SHARED((npad, K), F32),
            pltpu.SemaphoreType.DMA,
            pltpu.SemaphoreType.DMA,
        ],
    )
    def agg(src_hbm, dst_hbm, h_hbm, out_hbm, sidx0, sidx1, didx0, didx1,
            rows0, rows1, acc, gsem0, gsem1):
        c = lax.axis_index("c")
        s = lax.axis_index("s")
        sidx = (sidx0, sidx1)
        didx = (didx0, didx1)
        rows = (rows0, rows1)
        gsem = (gsem0, gsem1)
        zero16 = jnp.zeros((16,), F32)

        def fill(r, _):
            for j in range(K // 16):
                rows0[r, pl.ds(j * 16, 16)] = zero16
            return 0

        lax.fori_loop(0, ZR, fill, 0)
        for t in range(npw // ZR):
            pltpu.sync_copy(rows0, acc.at[pl.ds(s * npw + t * ZR, ZR)])
        plsc.subcore_barrier()

        base0 = s * ((m0 + m1) * EB) + c * (m0 * EB)

        def load_idx_issue_gather(j, slot):
            b = pl.multiple_of(base0 + j * EB, EB)
            pltpu.sync_copy(src_hbm.at[pl.ds(b, EB)], sidx[slot])
            pltpu.sync_copy(dst_hbm.at[pl.ds(b, EB)], didx[slot])
            pltpu.async_copy(h_hbm.at[sidx[slot]], rows[slot], gsem[slot])

        def wait_gather(slot):
            pltpu.make_async_copy(h_hbm.at[sidx[slot]], rows[slot], gsem[slot]).wait()

        def runloop(mch):
            load_idx_issue_gather(0, 0)

            def pair(i, _):
                j = 2 * i
                load_idx_issue_gather(j + 1, 1)
                wait_gather(0)
                pltpu.sync_copy(rows0, acc.at[didx0], add=True)
                load_idx_issue_gather(jnp.minimum(j + 2, mch - 1), 0)
                wait_gather(1)
                pltpu.sync_copy(rows1, acc.at[didx1], add=True)
                return 0

            lax.fori_loop(0, mch // 2, pair, 0)
            wait_gather(0)

        @pl.when(c == 0)
        def _():
            runloop(m0)

        @pl.when(c == 1)
        def _():
            runloop(m1)
        plsc.subcore_barrier()
        for t in range(npw // ZR):
            r = s * npw + t * ZR
            pltpu.sync_copy(acc.at[pl.ds(r, ZR)], out_hbm.at[pl.ds(c * npad + r, ZR)])

    return agg


_DOT = functools.partial(jnp.dot, precision=lax.Precision.HIGHEST)


def _stage_a(dp, x):
    npad, df = x.shape

    def body(dp_ref, x_ref, dv_ref, hs_ref):
        deg = dp_ref[0:npad, 0:1] + dp_ref[npad : 2 * npad, 0:1] + 1.0
        dv = jnp.broadcast_to(lax.rsqrt(deg), (npad, df))
        dv_ref[...] = dv
        hs_ref[...] = x_ref[...] * dv

    return pl.pallas_call(
        body,
        out_shape=(
            jax.ShapeDtypeStruct((npad, df), F32),
            jax.ShapeDtypeStruct((npad, df), F32),
        ),
    )(dp, x)


def _stage_b(a1, x, dv, W1, b1, W2):
    npad, df = x.shape
    h2w = W2.shape[1]

    def body(a_ref, x_ref, dv_ref, w1_ref, b1_ref, w2_ref, m2_ref, hs_ref):
        dv = dv_ref[...]
        z1 = dv * (a_ref[0:npad, :] + a_ref[npad : 2 * npad, :]) + dv * dv * x_ref[...]
        h1 = jnp.maximum(_DOT(z1, w1_ref[...]) + b1_ref[...], 0.0)
        m2 = _DOT(h1, w2_ref[...])
        m2_ref[...] = m2
        hs_ref[...] = m2 * dv

    return pl.pallas_call(
        body,
        out_shape=(
            jax.ShapeDtypeStruct((npad, h2w), F32),
            jax.ShapeDtypeStruct((npad, h2w), F32),
        ),
    )(a1, x, dv, W1, b1, W2)


def _stage_c(a2, m2, dv, b2, g1, be1, W3, n_real):
    npad, h2w = m2.shape
    h3w = W3.shape[1]

    def body(a_ref, m2_ref, dv_ref, b2_ref, g1_ref, be1_ref, w3_ref, m3_ref, hs_ref):
        dv = dv_ref[...]
        m2v = m2_ref[...]
        z2 = dv * (a_ref[0:npad, :] + a_ref[npad : 2 * npad, :]) + dv * dv * m2v + b2_ref[...]
        h2 = jnp.maximum(z2, 0.0)
        mask = (lax.broadcasted_iota(jnp.int32, (npad, h2w), 0) < n_real).astype(F32)
        mean = jnp.sum(h2 * mask, axis=0, keepdims=True) / n_real
        d = (h2 - mean) * mask
        var = jnp.sum(d * d, axis=0, keepdims=True) / n_real
        h2n = (h2 - mean) * lax.rsqrt(var + 1e-5) * g1_ref[...] + be1_ref[...]
        m3 = _DOT(h2n, w3_ref[...])
        m3_ref[...] = m3
        hs_ref[...] = jnp.concatenate(
            [m3 * dv_ref[0:npad, 0:h3w], jnp.zeros((npad, h2w - h3w), F32)], axis=1
        )

    return pl.pallas_call(
        body,
        out_shape=(
            jax.ShapeDtypeStruct((npad, h3w), F32),
            jax.ShapeDtypeStruct((npad, h2w), F32),
        ),
    )(a2, m2, dv, b2, g1, be1, W3)


def _stage_d(a3, m3, dv, b3, g2, be2, bT, fcW, fcb, n_real):
    npad, h3w = m3.shape
    g = bT.shape[0]
    c_out = fcW.shape[1]

    def body(a_ref, m3_ref, dv_ref, b3_ref, g2_ref, be2_ref, bT_ref, fcW_ref, fcb_ref, o_ref):
        dv = dv_ref[0:npad, 0:h3w]
        m3v = m3_ref[...]
        z3 = (dv * (a_ref[0:npad, 0:h3w] + a_ref[npad : 2 * npad, 0:h3w])
              + dv * dv * m3v + b3_ref[...])
        h3 = jnp.maximum(z3, 0.0)
        mask = (lax.broadcasted_iota(jnp.int32, (npad, h3w), 0) < n_real).astype(F32)
        mean = jnp.sum(h3 * mask, axis=0, keepdims=True) / n_real
        d = (h3 - mean) * mask
        var = jnp.sum(d * d, axis=0, keepdims=True) / n_real
        h3n = (h3 - mean) * lax.rsqrt(var + 1e-5) * g2_ref[...] + be2_ref[...]
        gid = lax.broadcasted_iota(jnp.int32, (g, npad), 0)
        MT = (bT_ref[...] == gid).astype(F32)
        seg = _DOT(MT, h3n)
        cnt = jnp.sum(MT, axis=1, keepdims=True)
        p = seg / jnp.maximum(cnt, 1.0)
        o_ref[...] = _DOT(p, fcW_ref[...]) + fcb_ref[...]

    return pl.pallas_call(
        body,
        out_shape=jax.ShapeDtypeStruct((g, c_out), F32),
    )(a3, m3, dv, b3, g2, be2, bT, fcW, fcb)


def kernel(x, edge_index, batch, W1, b1, W2, b2, g1, be1, W3, b3, g2, be2, fcW, fcb):
    n, df = x.shape
    e = edge_index.shape[1]
    g = 64
    h2w = W2.shape[1]
    h3w = W3.shape[1]

    npad = -(-n // (NS * ZR)) * (NS * ZR)
    cps = 2 * (-(-e // (EB * 2 * NS)))
    m0 = min(max(round(FRAC_C0 * cps / 2) * 2, 2), cps - 2)
    m1 = cps - m0
    epad = NS * cps * EB

    src = jnp.pad(edge_index[0], (0, epad - e), constant_values=n)
    dst = jnp.pad(edge_index[1], (0, epad - e), constant_values=n)
    x_p = _pad_rows(x, npad)
    bT = jnp.broadcast_to(
        jnp.pad(batch, (0, npad - n), constant_values=-1)[None, :], (g, npad)
    )

    aggs = {}

    def _agg(h):
        k = h.shape[1]
        if k not in aggs:
            aggs[k] = _agg_kernel(npad, m0, m1, k)
        return aggs[k](src, dst, h)

    deg_p = _deg_kernel(npad, m0, m1)(dst)
    dv, hs1 = _stage_a(deg_p, x_p)

    a1 = _agg(hs1)
    m2, hs2 = _stage_b(a1, x_p, dv, W1, b1.reshape(1, -1), W2)

    a2 = _agg(hs2)
    m3, hs3 = _stage_c(a2, m2, dv, b2.reshape(1, -1), g1.reshape(1, -1),
                       be1.reshape(1, -1), W3, n)

    a3 = _agg(hs3)
    out = _stage_d(a3, m3, dv, b3.reshape(1, -1), g2.reshape(1, -1),
                   be2.reshape(1, -1), bT, fcW, fcb.reshape(1, -1), n)
    return out
